# baseline (speedup 1.0000x reference)
import jax
import jax.numpy as jnp
from jax.experimental import pallas as pl
from jax.experimental.pallas import tpu as pltpu

TOP_K = 8
NUM_EXPERTS = 64
HIDDEN_DIM = 4096
TOKEN_BLOCK = 1024


def _router_block(hs_ref, wt_ref, logits_ref, topv_ref, topi_ref):
    x = hs_ref[...]
    wt = wt_ref[...]
    m_blk = x.shape[0]
    logits = jnp.dot(x, wt, preferred_element_type=jnp.float32)
    logits_ref[...] = logits
    r = jax.lax.broadcasted_iota(jnp.int32, (m_blk, m_blk), 0)
    c = jax.lax.broadcasted_iota(jnp.int32, (m_blk, m_blk), 1)
    eye = (r == c).astype(jnp.float32)
    lt = jax.lax.dot_general(
        logits, eye, (((0,), (0,)), ((), ())),
        preferred_element_type=jnp.float32,
    )

    eiota = jax.lax.broadcasted_iota(jnp.int32, (NUM_EXPERTS, m_blk), 0)
    work = lt
    vals, idxs = [], []
    for _ in range(TOP_K):
        m = jnp.max(work, axis=0, keepdims=True)
        idx = jnp.min(
            jnp.where(work == m, eiota, NUM_EXPERTS), axis=0, keepdims=True
        )
        vals.append(m)
        idxs.append(idx)
        work = jnp.where(eiota == idx, -jnp.inf, work)
    topv = jnp.concatenate(vals, axis=0)
    topi = jnp.concatenate(idxs, axis=0)

    e = jnp.exp(topv - topv[0:1, :])
    topv_ref[...] = e / jnp.sum(e, axis=0, keepdims=True)
    topi_ref[...] = topi


def kernel(hidden_states, weight):
    n_tokens = hidden_states.shape[0]
    blk = min(TOKEN_BLOCK, n_tokens)
    grid = (n_tokens // blk,)

    logits, topv_t, topi_t = pl.pallas_call(
        _router_block,
        grid=grid,
        in_specs=[
            pl.BlockSpec((blk, HIDDEN_DIM), lambda i: (i, 0)),
            pl.BlockSpec((HIDDEN_DIM, NUM_EXPERTS), lambda i: (0, 0)),
        ],
        out_specs=[
            pl.BlockSpec((blk, NUM_EXPERTS), lambda i: (i, 0)),
            pl.BlockSpec((TOP_K, blk), lambda i: (0, i)),
            pl.BlockSpec((TOP_K, blk), lambda i: (0, i)),
        ],
        out_shape=[
            jax.ShapeDtypeStruct((n_tokens, NUM_EXPERTS), jnp.float32),
            jax.ShapeDtypeStruct((TOP_K, n_tokens), jnp.float32),
            jax.ShapeDtypeStruct((TOP_K, n_tokens), jnp.int32),
        ],
        compiler_params=pltpu.CompilerParams(
            dimension_semantics=("arbitrary",),
        ),
    )(hidden_states, weight.T)
    return (logits, topv_t.T, topi_t.T)


# speedup vs baseline: 1.0300x; 1.0300x over previous
import jax
import jax.numpy as jnp
from jax.experimental import pallas as pl
from jax.experimental.pallas import tpu as pltpu

TOP_K = 8
NUM_EXPERTS = 64
HIDDEN_DIM = 4096
TOKEN_BLOCK = 1024


def _router_block(hs_ref, w_ref, logits_ref, topv_ref, topi_ref):
    x = hs_ref[...]
    w = w_ref[...]
    m_blk = x.shape[0]
    lt = jax.lax.dot_general(
        w, x, (((1,), (1,)), ((), ())), preferred_element_type=jnp.float32
    )
    r = jax.lax.broadcasted_iota(jnp.int32, (NUM_EXPERTS, NUM_EXPERTS), 0)
    c = jax.lax.broadcasted_iota(jnp.int32, (NUM_EXPERTS, NUM_EXPERTS), 1)
    eye = (r == c).astype(jnp.float32)
    logits_ref[...] = jax.lax.dot_general(
        lt, eye, (((0,), (0,)), ((), ())), preferred_element_type=jnp.float32
    )

    eiota = jax.lax.broadcasted_iota(jnp.int32, (NUM_EXPERTS, m_blk), 0)
    work = lt
    vals, idxs = [], []
    for _ in range(TOP_K):
        m = jnp.max(work, axis=0, keepdims=True)
        idx = jnp.min(
            jnp.where(work == m, eiota, NUM_EXPERTS), axis=0, keepdims=True
        )
        vals.append(m)
        idxs.append(idx)
        work = jnp.where(eiota == idx, -jnp.inf, work)
    topv = jnp.concatenate(vals, axis=0)
    topi = jnp.concatenate(idxs, axis=0)

    e = jnp.exp(topv - topv[0:1, :])
    topv_ref[...] = e / jnp.sum(e, axis=0, keepdims=True)
    topi_ref[...] = topi


def kernel(hidden_states, weight):
    n_tokens = hidden_states.shape[0]
    blk = min(TOKEN_BLOCK, n_tokens)
    grid = (n_tokens // blk,)

    logits, topv_t, topi_t = pl.pallas_call(
        _router_block,
        grid=grid,
        in_specs=[
            pl.BlockSpec((blk, HIDDEN_DIM), lambda i: (i, 0)),
            pl.BlockSpec((NUM_EXPERTS, HIDDEN_DIM), lambda i: (0, 0)),
        ],
        out_specs=[
            pl.BlockSpec((blk, NUM_EXPERTS), lambda i: (i, 0)),
            pl.BlockSpec((TOP_K, blk), lambda i: (0, i)),
            pl.BlockSpec((TOP_K, blk), lambda i: (0, i)),
        ],
        out_shape=[
            jax.ShapeDtypeStruct((n_tokens, NUM_EXPERTS), jnp.float32),
            jax.ShapeDtypeStruct((TOP_K, n_tokens), jnp.float32),
            jax.ShapeDtypeStruct((TOP_K, n_tokens), jnp.int32),
        ],
        compiler_params=pltpu.CompilerParams(
            dimension_semantics=("arbitrary",),
        ),
    )(hidden_states, weight)
    return (logits, topv_t.T, topi_t.T)
